# baseline (speedup 1.0000x reference)
import functools

import numpy as np

import jax
import jax.numpy as jnp
from jax import lax
from jax.experimental import pallas as pl
from jax.experimental.pallas import tpu as pltpu
from jax.experimental.pallas import tpu_sc as plsc

N_NODES = 10000
N_EDGES = 320000
D = 128
VD = 24

NC = 2
NS = 16
NW = NC * NS
CHUNK = 128
DW = D // 2
EPW = 10240
NCHUNK = EPW // CHUNK
E_PAD = EPW * NW
ACC_ROWS = 10112
ZROWS = ACC_ROWS // NS
OUT_RPS = 624
TRASH_ROW = N_NODES

_mesh = plsc.VectorSubcoreMesh(core_axis_name="c", subcore_axis_name="s")


NBUF = 2
SUP = 8
NSUP = NCHUNK // SUP


@functools.partial(
    pl.kernel,
    out_type=jax.ShapeDtypeStruct((NC, N_NODES, D), jnp.float32),
    mesh=_mesh,
    scratch_types=[
        pltpu.VMEM_SHARED((ACC_ROWS, D), jnp.float32),
        pltpu.VMEM((NCHUNK, CHUNK), jnp.int32),
        pltpu.VMEM((2, SUP, CHUNK), jnp.int32),
        pltpu.VMEM((NBUF, CHUNK, DW), jnp.int32),
        pltpu.VMEM((CHUNK, D), jnp.float32),
        pltpu.SemaphoreType.DMA((2,)),
        pltpu.SemaphoreType.DMA((NBUF,)),
        pltpu.SemaphoreType.DMA,
    ],
    compiler_params=pltpu.CompilerParams(use_tc_tiling_on_sc=False),
)
def _segsum_sc(mu_hbm, src_hbm, dst_hbm, zeros_hbm, out_hbm,
               acc, src_v, dst_v, rows_bf, rows_f, isem_d, gsem, ssem):
    c = lax.axis_index("c")
    s = lax.axis_index("s")
    wid = s * NC + c

    def start_idx(u, m):
        pltpu.async_copy(dst_hbm.at[pl.ds(wid * NCHUNK + u * SUP, SUP)],
                         dst_v.at[m], isem_d.at[m])

    def wait_idx(u, m):
        pltpu.make_async_copy(dst_hbm.at[pl.ds(wid * NCHUNK + u * SUP, SUP)],
                              dst_v.at[m], isem_d.at[m]).wait()

    def start_gather(k, b):
        pltpu.async_copy(mu_hbm.at[src_v.at[k]], rows_bf.at[b], gsem.at[b])

    def wait_gather(k, b):
        pltpu.make_async_copy(mu_hbm.at[src_v.at[k]], rows_bf.at[b],
                              gsem.at[b]).wait()

    def start_scatter(k):
        pltpu.async_copy(rows_f, acc.at[dst_v.at[(k // SUP) % 2, k % SUP]],
                         ssem, add=True)

    def wait_scatter(k):
        pltpu.make_async_copy(rows_f, acc.at[dst_v.at[(k // SUP) % 2, k % SUP]],
                              ssem).wait()

    hi_mask = jnp.full((16,), -65536, jnp.int32)

    def convert(b):
        def row(r, carry):
            for h in range(4):
                w = rows_bf[b, r, pl.ds(16 * h, 16)]
                lo = lax.bitcast_convert_type(w << 16, jnp.float32)
                hi = lax.bitcast_convert_type(w & hi_mask, jnp.float32)
                rows_f[r, pl.ds(32 * h, 16)] = lo
                rows_f[r, pl.ds(32 * h + 16, 16)] = hi
            return carry

        lax.fori_loop(0, CHUNK, row, 0)

    start_idx(0, 0)
    start_idx(1, 1)
    pltpu.sync_copy(src_hbm.at[pl.ds(wid * NCHUNK, NCHUNK)], src_v)
    pltpu.sync_copy(zeros_hbm, acc.at[pl.ds(s * ZROWS, ZROWS)])
    plsc.subcore_barrier()
    start_gather(0, 0)
    start_gather(1, 1)
    wait_idx(0, 0)
    wait_gather(0, 0)
    convert(0)
    start_scatter(0)
    start_gather(2, 0)

    def body(k, carry):
        b = lax.rem(k, NBUF)
        u = k // SUP
        kmod = lax.rem(k, SUP)
        wait_gather(k, b)
        wait_scatter(k - 1)

        @pl.when(kmod == 0)
        def _():
            wait_idx(u, lax.rem(u, 2))

        @pl.when((kmod == 0) & (u <= NSUP - 2))
        def _():
            start_idx(u + 1, lax.rem(u + 1, 2))

        convert(b)
        start_scatter(k)

        @pl.when(k + NBUF < NCHUNK)
        def _():
            start_gather(k + NBUF, b)

        return carry

    lax.fori_loop(1, NCHUNK, body, 0)
    wait_scatter(NCHUNK - 1)
    plsc.subcore_barrier()

    @pl.when(s < NS - 1)
    def _copy_main():
        pltpu.sync_copy(acc.at[pl.ds(s * OUT_RPS, OUT_RPS)],
                        out_hbm.at[c, pl.ds(s * OUT_RPS, OUT_RPS)])

    @pl.when(s == NS - 1)
    def _copy_tail():
        tail = N_NODES - (NS - 1) * OUT_RPS
        pltpu.sync_copy(acc.at[pl.ds((NS - 1) * OUT_RPS, tail)],
                        out_hbm.at[c, pl.ds((NS - 1) * OUT_RPS, tail)])


def _dense_body(x_ref, w1t_ref, p0_ref, p1_ref, w2t_ref, o_ref):
    xh = jnp.dot(x_ref[...], w1t_ref[...], preferred_element_type=jnp.float32)
    agg = jnp.dot(p0_ref[...] + p1_ref[...], w2t_ref[...],
                  preferred_element_type=jnp.float32)
    o_ref[...] = jnp.maximum(xh + agg, 0.0)


_ROWS_BLK = 1000

_dense = pl.pallas_call(
    _dense_body,
    grid=(N_NODES // _ROWS_BLK,),
    in_specs=[
        pl.BlockSpec((_ROWS_BLK, VD), lambda i: (i, 0)),
        pl.BlockSpec((VD, D), lambda i: (0, 0)),
        pl.BlockSpec((_ROWS_BLK, D), lambda i: (i, 0)),
        pl.BlockSpec((_ROWS_BLK, D), lambda i: (i, 0)),
        pl.BlockSpec((D, D), lambda i: (0, 0)),
    ],
    out_specs=pl.BlockSpec((_ROWS_BLK, D), lambda i: (i, 0)),
    out_shape=jax.ShapeDtypeStruct((N_NODES, D), jnp.float32),
)


_SIGMA = np.empty((D,), np.int32)
for _g in range(4):
    _SIGMA[32 * _g + np.arange(16)] = 32 * _g + 2 * np.arange(16)
    _SIGMA[32 * _g + 16 + np.arange(16)] = 32 * _g + 2 * np.arange(16) + 1


def kernel(mu, x, edge_index, W1, W2):
    ei = edge_index.astype(jnp.int32)
    pad = E_PAD - N_EDGES
    src_p = jnp.concatenate([ei[1], jnp.zeros((pad,), jnp.int32)])
    src_p = src_p.reshape(E_PAD // CHUNK, CHUNK)
    trash = TRASH_ROW + jnp.arange(pad, dtype=jnp.int32) % (ACC_ROWS - N_NODES)
    dst_p = jnp.concatenate([ei[0], trash])
    dst_p = dst_p.reshape(E_PAD // CHUNK, CHUNK)
    zeros = jnp.zeros((ZROWS, D), jnp.float32)
    mbf = mu.astype(jnp.bfloat16).reshape(N_NODES, DW, 2)
    mu_pk = jax.lax.bitcast_convert_type(mbf, jnp.int32)
    partials = _segsum_sc(mu_pk, src_p, dst_p, zeros)
    w2tp = W2.T[jnp.asarray(_SIGMA)]
    return _dense(x, W1.T, partials[0], partials[1], w2tp)


# speedup vs baseline: 1.0492x; 1.0492x over previous
import functools

import numpy as np

import jax
import jax.numpy as jnp
from jax import lax
from jax.experimental import pallas as pl
from jax.experimental.pallas import tpu as pltpu
from jax.experimental.pallas import tpu_sc as plsc

N_NODES = 10000
N_EDGES = 320000
D = 128
VD = 24

NC = 2
NS = 16
NW = NC * NS
CHUNK = 128
DW = D // 2
EPW = 10240
NCHUNK = EPW // CHUNK
E_PAD = EPW * NW
ACC_ROWS = 10112
ZROWS = ACC_ROWS // NS
OUT_RPS = 624
TRASH_ROW = N_NODES

_mesh = plsc.VectorSubcoreMesh(core_axis_name="c", subcore_axis_name="s")


NBUF = 2
SUP = 8
NSUP = NCHUNK // SUP


@functools.partial(
    pl.kernel,
    out_type=jax.ShapeDtypeStruct((NC, N_NODES, D), jnp.float32),
    mesh=_mesh,
    scratch_types=[
        pltpu.VMEM_SHARED((ACC_ROWS, D), jnp.float32),
        pltpu.VMEM((NCHUNK, CHUNK), jnp.int32),
        pltpu.VMEM((2, SUP, CHUNK), jnp.int32),
        pltpu.VMEM((NBUF, CHUNK, DW), jnp.int32),
        pltpu.VMEM((CHUNK, D), jnp.float32),
        pltpu.SemaphoreType.DMA((2,)),
        pltpu.SemaphoreType.DMA((NBUF,)),
        pltpu.SemaphoreType.DMA,
    ],
    compiler_params=pltpu.CompilerParams(use_tc_tiling_on_sc=False),
)
def _segsum_sc(mu_hbm, src_hbm, dst_hbm, zeros_hbm, out_hbm,
               acc, src_v, dst_v, rows_bf, rows_f, isem_d, gsem, ssem):
    c = lax.axis_index("c")
    s = lax.axis_index("s")
    wid = s * NC + c

    def start_idx(u, m):
        pltpu.async_copy(dst_hbm.at[pl.ds(wid * NCHUNK + u * SUP, SUP)],
                         dst_v.at[m], isem_d.at[m])

    def wait_idx(u, m):
        pltpu.make_async_copy(dst_hbm.at[pl.ds(wid * NCHUNK + u * SUP, SUP)],
                              dst_v.at[m], isem_d.at[m]).wait()

    def start_gather(k, b):
        pltpu.async_copy(mu_hbm.at[src_v.at[k]], rows_bf.at[b], gsem.at[b])

    def wait_gather(k, b):
        pltpu.make_async_copy(mu_hbm.at[src_v.at[k]], rows_bf.at[b],
                              gsem.at[b]).wait()

    def start_scatter(k):
        pltpu.async_copy(rows_f, acc.at[dst_v.at[(k // SUP) % 2, k % SUP]],
                         ssem, add=True)

    def wait_scatter(k):
        pltpu.make_async_copy(rows_f, acc.at[dst_v.at[(k // SUP) % 2, k % SUP]],
                              ssem).wait()

    hi_mask = jnp.full((16,), -65536, jnp.int32)

    def convert(b):
        def row(r, carry):
            for h in range(4):
                w = rows_bf[b, r, pl.ds(16 * h, 16)]
                lo = lax.bitcast_convert_type(w << 16, jnp.float32)
                hi = lax.bitcast_convert_type(w & hi_mask, jnp.float32)
                rows_f[r, pl.ds(32 * h, 16)] = lo
                rows_f[r, pl.ds(32 * h + 16, 16)] = hi
            return carry

        lax.fori_loop(0, CHUNK, row, 0)

    start_idx(0, 0)
    start_idx(1, 1)
    pltpu.sync_copy(src_hbm.at[pl.ds(wid * NCHUNK, NCHUNK)], src_v)
    pltpu.sync_copy(zeros_hbm, acc.at[pl.ds(s * ZROWS, ZROWS)])
    plsc.subcore_barrier()
    start_gather(0, 0)
    start_gather(1, 1)
    wait_idx(0, 0)
    wait_gather(0, 0)
    convert(0)
    start_scatter(0)
    start_gather(2, 0)

    def body(k, carry):
        b = lax.rem(k, NBUF)
        u = k // SUP
        kmod = lax.rem(k, SUP)
        wait_gather(k, b)
        wait_scatter(k - 1)

        @pl.when(kmod == 0)
        def _():
            wait_idx(u, lax.rem(u, 2))

        @pl.when((kmod == 0) & (u <= NSUP - 2))
        def _():
            start_idx(u + 1, lax.rem(u + 1, 2))

        convert(b)
        start_scatter(k)

        @pl.when(k + NBUF < NCHUNK)
        def _():
            start_gather(k + NBUF, b)

        return carry

    lax.fori_loop(1, NCHUNK, body, 0)
    wait_scatter(NCHUNK - 1)
    plsc.subcore_barrier()

    @pl.when(s < NS - 1)
    def _copy_main():
        pltpu.sync_copy(acc.at[pl.ds(s * OUT_RPS, OUT_RPS)],
                        out_hbm.at[c, pl.ds(s * OUT_RPS, OUT_RPS)])

    @pl.when(s == NS - 1)
    def _copy_tail():
        tail = N_NODES - (NS - 1) * OUT_RPS
        pltpu.sync_copy(acc.at[pl.ds((NS - 1) * OUT_RPS, tail)],
                        out_hbm.at[c, pl.ds((NS - 1) * OUT_RPS, tail)])


def _dense_body(x_ref, w1t_ref, p0_ref, p1_ref, w2t_ref, o_ref):
    xh = jnp.dot(x_ref[...], w1t_ref[...], preferred_element_type=jnp.float32)
    agg = jnp.dot(p0_ref[...] + p1_ref[...], w2t_ref[...],
                  preferred_element_type=jnp.float32)
    o_ref[...] = jnp.maximum(xh + agg, 0.0)


_ROWS_BLK = 1000

_dense = pl.pallas_call(
    _dense_body,
    grid=(N_NODES // _ROWS_BLK,),
    in_specs=[
        pl.BlockSpec((_ROWS_BLK, VD), lambda i: (i, 0)),
        pl.BlockSpec((VD, D), lambda i: (0, 0)),
        pl.BlockSpec((_ROWS_BLK, D), lambda i: (i, 0)),
        pl.BlockSpec((_ROWS_BLK, D), lambda i: (i, 0)),
        pl.BlockSpec((D, D), lambda i: (0, 0)),
    ],
    out_specs=pl.BlockSpec((_ROWS_BLK, D), lambda i: (i, 0)),
    out_shape=jax.ShapeDtypeStruct((N_NODES, D), jnp.float32),
)


_SIGMA = np.empty((D,), np.int32)
for _g in range(4):
    _SIGMA[32 * _g + np.arange(16)] = 16 * _g + np.arange(16)
    _SIGMA[32 * _g + 16 + np.arange(16)] = 16 * _g + 64 + np.arange(16)


def kernel(mu, x, edge_index, W1, W2):
    ei = edge_index.astype(jnp.int32)
    pad = E_PAD - N_EDGES
    src_p = jnp.concatenate([ei[1], jnp.zeros((pad,), jnp.int32)])
    src_p = src_p.reshape(E_PAD // CHUNK, CHUNK)
    trash = TRASH_ROW + jnp.arange(pad, dtype=jnp.int32) % (ACC_ROWS - N_NODES)
    dst_p = jnp.concatenate([ei[0], trash])
    dst_p = dst_p.reshape(E_PAD // CHUNK, CHUNK)
    zeros = jnp.zeros((ZROWS, D), jnp.float32)
    mu_i = jax.lax.bitcast_convert_type(mu, jnp.int32) + 0x8000
    lo_w = (mu_i[:, :DW] >> 16) & 0xFFFF
    hi_w = mu_i[:, DW:] & jnp.int32(-65536)
    mu_pk = lo_w | hi_w
    partials = _segsum_sc(mu_pk, src_p, dst_p, zeros)
    w2tp = W2.T[jnp.asarray(_SIGMA)]
    return _dense(x, W1.T, partials[0], partials[1], w2tp)


# speedup vs baseline: 1.1692x; 1.1144x over previous
import functools

import numpy as np

import jax
import jax.numpy as jnp
from jax import lax
from jax.experimental import pallas as pl
from jax.experimental.pallas import tpu as pltpu
from jax.experimental.pallas import tpu_sc as plsc

N_NODES = 10000
N_EDGES = 320000
D = 128
VD = 24

NC = 2
NS = 16
NW = NC * NS
CHUNK = 128
DW = D // 2
NCH_ALL = N_EDGES // CHUNK
WCHUNK = 80
LCHUNK = NCH_ALL - (NW - 1) * WCHUNK
ACC_ROWS = 10112
ZROWS = ACC_ROWS // NS
OUT_RPS = 624

_mesh = plsc.VectorSubcoreMesh(core_axis_name="c", subcore_axis_name="s")


NBUF = 2
SUP = 4


@functools.partial(
    pl.kernel,
    out_type=(jax.ShapeDtypeStruct((N_NODES, D), jnp.float32),
              jax.ShapeDtypeStruct((N_NODES, D), jnp.float32)),
    mesh=_mesh,
    scratch_types=[
        pltpu.VMEM_SHARED((ACC_ROWS, D), jnp.float32),
        pltpu.VMEM((WCHUNK, CHUNK), jnp.int32),
        pltpu.VMEM((2, SUP, CHUNK), jnp.int32),
        pltpu.VMEM((NBUF, CHUNK, DW), jnp.int32),
        pltpu.VMEM((CHUNK, D), jnp.float32),
        pltpu.SemaphoreType.DMA((2,)),
        pltpu.SemaphoreType.DMA((NBUF,)),
        pltpu.SemaphoreType.DMA,
    ],
    compiler_params=pltpu.CompilerParams(use_tc_tiling_on_sc=False),
)
def _segsum_sc(mu_hbm, ep_hbm, zeros_hbm, out0_hbm, out1_hbm,
               acc, src_v, dst_v, rows_bf, rows_f, isem_d, gsem, ssem):
    c = lax.axis_index("c")
    s = lax.axis_index("s")
    wid = s * NC + c
    k0 = wid * WCHUNK
    islast = wid == NW - 1
    nch = jnp.where(islast, LCHUNK, WCHUNK)
    nsup = jnp.where(islast, LCHUNK // SUP, WCHUNK // SUP)

    def start_idx(u, m):
        pltpu.async_copy(ep_hbm.at[0, pl.ds(k0 + u * SUP, SUP)], dst_v.at[m],
                         isem_d.at[m])

    def wait_idx(u, m):
        pltpu.make_async_copy(ep_hbm.at[0, pl.ds(k0 + u * SUP, SUP)],
                              dst_v.at[m], isem_d.at[m]).wait()

    def start_gather(k, b):
        pltpu.async_copy(mu_hbm.at[src_v.at[k]], rows_bf.at[b], gsem.at[b])

    def wait_gather(k, b):
        pltpu.make_async_copy(mu_hbm.at[src_v.at[k]], rows_bf.at[b],
                              gsem.at[b]).wait()

    def start_scatter(k):
        pltpu.async_copy(rows_f, acc.at[dst_v.at[(k // SUP) % 2, k % SUP]],
                         ssem, add=True)

    def wait_scatter(k):
        pltpu.make_async_copy(rows_f, acc.at[dst_v.at[(k // SUP) % 2, k % SUP]],
                              ssem).wait()

    hi_mask = jnp.full((16,), -65536, jnp.int32)

    def convert(b):
        def row(r, carry):
            for h in range(4):
                w = rows_bf[b, r, pl.ds(16 * h, 16)]
                lo = lax.bitcast_convert_type(w << 16, jnp.float32)
                hi = lax.bitcast_convert_type(w & hi_mask, jnp.float32)
                rows_f[r, pl.ds(32 * h, 16)] = lo
                rows_f[r, pl.ds(32 * h + 16, 16)] = hi
            return carry

        lax.fori_loop(0, CHUNK, row, 0)

    start_idx(0, 0)
    start_idx(1, 1)

    @pl.when(jnp.logical_not(islast))
    def _():
        pltpu.sync_copy(ep_hbm.at[1, pl.ds(k0, WCHUNK)], src_v)

    @pl.when(islast)
    def _():
        pltpu.sync_copy(ep_hbm.at[1, pl.ds(k0, LCHUNK)],
                        src_v.at[pl.ds(0, LCHUNK)])

    pltpu.sync_copy(zeros_hbm, acc.at[pl.ds(s * ZROWS, ZROWS)])
    plsc.subcore_barrier()
    start_gather(0, 0)
    start_gather(1, 1)
    wait_idx(0, 0)
    wait_gather(0, 0)
    convert(0)
    start_scatter(0)
    start_gather(2, 0)

    def body(k, carry):
        b = lax.rem(k, NBUF)
        u = k // SUP
        kmod = lax.rem(k, SUP)
        wait_gather(k, b)
        wait_scatter(k - 1)

        @pl.when(kmod == 0)
        def _():
            wait_idx(u, lax.rem(u, 2))

        @pl.when((kmod == 0) & (u <= nsup - 2))
        def _():
            start_idx(u + 1, lax.rem(u + 1, 2))

        convert(b)
        start_scatter(k)

        @pl.when(k + NBUF < nch)
        def _():
            start_gather(k + NBUF, b)

        return carry

    lax.fori_loop(1, nch, body, 0)
    wait_scatter(nch - 1)
    plsc.subcore_barrier()

    for cc, out_hbm in ((0, out0_hbm), (1, out1_hbm)):
        @pl.when((c == cc) & (s < NS - 1))
        def _(out_hbm=out_hbm):
            pltpu.sync_copy(acc.at[pl.ds(s * OUT_RPS, OUT_RPS)],
                            out_hbm.at[pl.ds(s * OUT_RPS, OUT_RPS)])

        @pl.when((c == cc) & (s == NS - 1))
        def _(out_hbm=out_hbm):
            tail = N_NODES - (NS - 1) * OUT_RPS
            pltpu.sync_copy(acc.at[pl.ds((NS - 1) * OUT_RPS, tail)],
                            out_hbm.at[pl.ds((NS - 1) * OUT_RPS, tail)])


def _dense_body(x_ref, w1t_ref, p0_ref, p1_ref, w2t_ref, o_ref):
    xh = jnp.dot(x_ref[...], w1t_ref[...], preferred_element_type=jnp.float32)
    agg = jnp.dot(p0_ref[...] + p1_ref[...], w2t_ref[...],
                  preferred_element_type=jnp.float32)
    o_ref[...] = jnp.maximum(xh + agg, 0.0)


_ROWS_BLK = 1000

_dense = pl.pallas_call(
    _dense_body,
    grid=(N_NODES // _ROWS_BLK,),
    in_specs=[
        pl.BlockSpec((_ROWS_BLK, VD), lambda i: (i, 0)),
        pl.BlockSpec((VD, D), lambda i: (0, 0)),
        pl.BlockSpec((_ROWS_BLK, D), lambda i: (i, 0)),
        pl.BlockSpec((_ROWS_BLK, D), lambda i: (i, 0)),
        pl.BlockSpec((D, D), lambda i: (0, 0)),
    ],
    out_specs=pl.BlockSpec((_ROWS_BLK, D), lambda i: (i, 0)),
    out_shape=jax.ShapeDtypeStruct((N_NODES, D), jnp.float32),
)


_SIGMA = np.empty((D,), np.int32)
for _g in range(4):
    _SIGMA[32 * _g + np.arange(16)] = 16 * _g + np.arange(16)
    _SIGMA[32 * _g + 16 + np.arange(16)] = 16 * _g + 64 + np.arange(16)


def kernel(mu, x, edge_index, W1, W2):
    ep3 = edge_index.astype(jnp.int32).reshape(2, NCH_ALL, CHUNK)
    zeros = jnp.zeros((ZROWS, D), jnp.float32)
    mu_i = jax.lax.bitcast_convert_type(mu, jnp.int32) + 0x8000
    lo_w = (mu_i[:, :DW] >> 16) & 0xFFFF
    hi_w = mu_i[:, DW:] & jnp.int32(-65536)
    mu_pk = lo_w | hi_w
    p0, p1 = _segsum_sc(mu_pk, ep3, zeros)
    w2tp = W2.T[jnp.asarray(_SIGMA)]
    return _dense(x, W1.T, p0, p1, w2tp)
